# speedup vs baseline: 5.0327x; 5.0327x over previous
import jax
import jax.numpy as jnp
from jax.experimental import pallas as pl


def _body(tab_ref, out_ref):
    out_ref[...] = jnp.broadcast_to(tab_ref[...][None, :, :], out_ref.shape)


def kernel(x, table):
    B, T = x.shape
    _, D = table.shape
    TBLK = 512
    out = pl.pallas_call(
        _body,
        grid=(T // TBLK,),
        in_specs=[pl.BlockSpec((TBLK, D), lambda i: (i, 0))],
        out_specs=pl.BlockSpec((B, TBLK, D), lambda i: (0, i, 0)),
        out_shape=jax.ShapeDtypeStruct((B, T, D), jnp.float32),
    )(table)
    return out


# speedup vs baseline: 5.1806x; 1.0294x over previous
import jax
import jax.numpy as jnp
from jax.experimental import pallas as pl


def _body(tab_ref, out_ref):
    out_ref[...] = jnp.broadcast_to(tab_ref[...][None, :, :], out_ref.shape)


def kernel(x, table):
    B, T = x.shape
    _, D = table.shape
    TBLK = 1024
    out = pl.pallas_call(
        _body,
        grid=(T // TBLK,),
        in_specs=[pl.BlockSpec((TBLK, D), lambda i: (i, 0))],
        out_specs=pl.BlockSpec((B, TBLK, D), lambda i: (0, i, 0)),
        out_shape=jax.ShapeDtypeStruct((B, T, D), jnp.float32),
    )(table)
    return out
